# baseline (speedup 1.0000x reference)
import functools

import jax
import jax.numpy as jnp
from jax import lax
from jax.experimental import pallas as pl
from jax.experimental.pallas import tpu as pltpu
from jax.experimental.pallas import tpu_sc as plsc

_SLOPE = (1.0 / 8.0 + 1.0 / 3.0) / 2.0

_NC = 2
_NS = 16
_LANES = 16


def _gru_body(L, T, wi_ref, wg_ref, ug_ref, bg_ref, q_ref):
    def sigmoid(v):
        return 1.0 / (1.0 + jnp.exp(-v))

    for l in range(L):
        Q = wi_ref[l]
        for t in range(T):
            z = Q
            upd = sigmoid(
                jnp.dot(wg_ref[l, 0], z, preferred_element_type=jnp.float32)
                + jnp.dot(ug_ref[l, 0], Q, preferred_element_type=jnp.float32)
                + bg_ref[l, 0])
            rst = sigmoid(
                jnp.dot(wg_ref[l, 1], z, preferred_element_type=jnp.float32)
                + jnp.dot(ug_ref[l, 1], Q, preferred_element_type=jnp.float32)
                + bg_ref[l, 1])
            hcap = jnp.tanh(
                jnp.dot(wg_ref[l, 2], z, preferred_element_type=jnp.float32)
                + jnp.dot(ug_ref[l, 2], rst * Q, preferred_element_type=jnp.float32)
                + bg_ref[l, 2])
            Q = (1.0 - upd) * Q + upd * hcap
            q_ref[l, t] = Q


def _evolve_weights(W_init, Wg, Ug, bg, T):
    L, D, _ = W_init.shape
    return pl.pallas_call(
        functools.partial(_gru_body, L, T),
        out_shape=jax.ShapeDtypeStruct((L, T, D, D), jnp.float32),
    )(W_init, Wg, Ug, bg)


def _mm_body(x_ref, q_ref, o_ref):
    o_ref[0] = jnp.dot(x_ref[0], q_ref[0], preferred_element_type=jnp.float32)


def _matmul_t(x, q):
    T, N, D = x.shape
    return pl.pallas_call(
        _mm_body,
        grid=(T,),
        in_specs=[
            pl.BlockSpec((1, N, D), lambda t: (t, 0, 0)),
            pl.BlockSpec((1, D, D), lambda t: (t, 0, 0)),
        ],
        out_specs=pl.BlockSpec((1, N, D), lambda t: (t, 0, 0)),
        out_shape=jax.ShapeDtypeStruct((T, N, D), jnp.float32),
    )(x, q)


def _combine_mm_body(p_ref, q_ref, o_ref):
    s = p_ref[0] + p_ref[1]
    a = jnp.where(s >= 0, s, s * _SLOPE)
    o_ref[...] = jnp.dot(a, q_ref[...], preferred_element_type=jnp.float32)


def _combine_matmul(partials, q):
    _, N, D = partials.shape
    NB = 5
    BN = N // NB
    return pl.pallas_call(
        _combine_mm_body,
        grid=(NB,),
        in_specs=[
            pl.BlockSpec((2, BN, D), lambda nb: (0, nb, 0)),
            pl.BlockSpec((D, D), lambda nb: (0, 0)),
        ],
        out_specs=pl.BlockSpec((BN, D), lambda nb: (nb, 0)),
        out_shape=jax.ShapeDtypeStruct((N, D), jnp.float32),
    )(partials, q)


def _final_body(p_ref, o_ref):
    s = p_ref[0] + p_ref[1]
    o_ref[...] = jnp.where(s >= 0, s, s * _SLOPE)


def _final_act(partials):
    _, N, D = partials.shape
    NB = 5
    BN = N // NB
    return pl.pallas_call(
        _final_body,
        grid=(NB,),
        in_specs=[pl.BlockSpec((2, BN, D), lambda nb: (0, nb, 0))],
        out_specs=pl.BlockSpec((BN, D), lambda nb: (nb, 0)),
        out_shape=jax.ShapeDtypeStruct((N, D), jnp.float32),
    )(partials)


def _make_sc_layer(T, N, E, D):
    NW = _NC * _NS
    EPT = E // NW
    K = 128
    NCHUNK = 79
    EPP = NCHUNK * K
    MAIN = NCHUNK - 1
    RPT = 624
    LAST = N - (_NS - 1) * RPT
    assert EPP >= EPT and MAIN % 3 == 0 and RPT % 8 == 0 and LAST % 8 == 0

    mesh = plsc.VectorSubcoreMesh(core_axis_name="c", subcore_axis_name="s")

    @functools.partial(
        pl.kernel,
        mesh=mesh,
        out_type=jax.ShapeDtypeStruct((_NC, N, D), jnp.float32),
        scratch_types=[
            pltpu.VMEM((K,), jnp.int32),
            pltpu.VMEM((K,), jnp.int32),
            pltpu.VMEM((K,), jnp.int32),
            pltpu.VMEM((K,), jnp.int32),
            pltpu.VMEM((K,), jnp.int32),
            pltpu.VMEM((K,), jnp.int32),
            pltpu.VMEM((K,), jnp.float32),
            pltpu.VMEM((K,), jnp.float32),
            pltpu.VMEM((K,), jnp.float32),
            pltpu.VMEM((K,), jnp.int32),
            pltpu.VMEM((K,), jnp.int32),
            pltpu.VMEM((K,), jnp.int32),
            pltpu.VMEM((K, D), jnp.float32),
            pltpu.VMEM((K, D), jnp.float32),
            pltpu.VMEM((K, D), jnp.float32),
            pltpu.VMEM_SHARED((N, D), jnp.float32),
            pltpu.SemaphoreType.DMA,
            pltpu.SemaphoreType.DMA,
            pltpu.SemaphoreType.DMA,
            pltpu.SemaphoreType.DMA,
            pltpu.SemaphoreType.DMA,
            pltpu.SemaphoreType.DMA,
            pltpu.SemaphoreType.DMA,
            pltpu.SemaphoreType.DMA,
            pltpu.SemaphoreType.DMA,
        ],
    )
    def sc_layer(h, col_hbm, row_hbm, w_hbm, out_hbm,
                 cb0, cb1, cb2, rb0, rb1, rb2, wb0, wb1, wb2,
                 srb0, srb1, srb2, rows0, rows1, rows2, acc,
                 g0, g1, g2, i0, i1, i2, s0, s1, s2):
        c = lax.axis_index("c")
        s = lax.axis_index("s")
        wid = c * _NS + s
        base_e = wid * EPP
        CB = (cb0, cb1, cb2)
        RB = (rb0, rb1, rb2)
        WB = (wb0, wb1, wb2)
        SRB = (srb0, srb1, srb2)
        ROWS = (rows0, rows1, rows2)
        GS = (g0, g1, g2)
        IS = (i0, i1, i2)
        SS = (s0, s1, s2)
        zero = jnp.zeros((_LANES,), jnp.float32)

        def issue_idx(i, b, sync):
            e0 = pl.multiple_of(base_e + i * K, 8)
            if sync:
                pltpu.sync_copy(col_hbm.at[pl.ds(e0, K)], CB[b])
                pltpu.sync_copy(row_hbm.at[pl.ds(e0, K)], RB[b])
                pltpu.sync_copy(w_hbm.at[pl.ds(e0, K)], WB[b])
            else:
                pltpu.async_copy(col_hbm.at[pl.ds(e0, K)], CB[b], IS[b])
                pltpu.async_copy(row_hbm.at[pl.ds(e0, K)], RB[b], IS[b])
                pltpu.async_copy(w_hbm.at[pl.ds(e0, K)], WB[b], IS[b])

        def wait_idx(i, b):
            e0 = pl.multiple_of(base_e + i * K, 8)
            pltpu.make_async_copy(col_hbm.at[pl.ds(e0, K)], CB[b], IS[b]).wait()
            pltpu.make_async_copy(row_hbm.at[pl.ds(e0, K)], RB[b], IS[b]).wait()
            pltpu.make_async_copy(w_hbm.at[pl.ds(e0, K)], WB[b], IS[b]).wait()

        def issue_gather(b):
            pltpu.async_copy(h.at[CB[b]], ROWS[b], GS[b])

        def drain_scatter(b):
            pltpu.make_async_copy(ROWS[b], acc.at[SRB[b]], SS[b]).wait()

        def finish_chunk(b):
            pltpu.make_async_copy(h.at[CB[b]], ROWS[b], GS[b]).wait()
            rbuf = ROWS[b]

            for g in range(K // _LANES):
                sl = pl.ds(g * _LANES, _LANES)
                SRB[b][sl] = RB[b][sl]

            def scale(g, _):
                wg = WB[b][pl.ds(g * _LANES, _LANES)]
                for jj in range(_LANES):
                    wj = wg[jj]
                    r = g * _LANES + jj
                    for d in range(D // _LANES):
                        sl = pl.ds(d * _LANES, _LANES)
                        rbuf[r, sl] = rbuf[r, sl] * wj
                return 0

            lax.fori_loop(0, K // _LANES, scale, 0, unroll=2)
            pltpu.async_copy(rbuf, acc.at[SRB[b]], SS[b], add=True)

        if True:
            def zfill(r, _):
                for d in range(D // _LANES):
                    rows0[r, pl.ds(d * _LANES, _LANES)] = zero
                return 0

            lax.fori_loop(0, K, zfill, 0)
            for off in range(0, RPT - RPT % K, K):
                pltpu.sync_copy(rows0, acc.at[pl.ds(s * RPT + off, K)])
            if RPT % K:
                pltpu.sync_copy(
                    rows0.at[pl.ds(0, RPT % K)],
                    acc.at[pl.ds(s * RPT + RPT - RPT % K, RPT % K)])

            @pl.when(s == _NS - 1)
            def _():
                pltpu.sync_copy(rows0.at[pl.ds(0, LAST - RPT)],
                                acc.at[pl.ds(N - (LAST - RPT), LAST - RPT)])

            plsc.subcore_barrier()

            issue_idx(0, 0, True)
            issue_gather(0)
            issue_idx(1, 1, True)
            issue_gather(1)
            issue_idx(2, 2, False)

            def triple(p, _):
                for u in range(3):
                    i = p * 3 + u
                    nb = (u + 2) % 3
                    @pl.when(i + 2 < NCHUNK)
                    def _():
                        wait_idx(i + 2, nb)

                    @pl.when(i >= 1)
                    def _():
                        drain_scatter(nb)

                    @pl.when(i + 2 < NCHUNK)
                    def _():
                        issue_gather(nb)

                    finish_chunk(u)
                    nxt = i + 3

                    @pl.when(nxt < NCHUNK)
                    def _():
                        issue_idx(nxt, u, False)
                return 0

            lax.fori_loop(0, MAIN // 3, triple, 0)
            finish_chunk(MAIN % 3)
            drain_scatter((MAIN - 1) % 3)
            drain_scatter(MAIN % 3)
            plsc.subcore_barrier()

            pltpu.sync_copy(acc.at[pl.ds(s * RPT, RPT)],
                            out_hbm.at[c, pl.ds(s * RPT, RPT)])

            @pl.when(s == _NS - 1)
            def _():
                pltpu.sync_copy(acc.at[pl.ds(N - (LAST - RPT), LAST - RPT)],
                                out_hbm.at[c, pl.ds(N - (LAST - RPT),
                                                    LAST - RPT)])

            plsc.subcore_barrier()

    def pad_edges(col, row, w):
        pad = EPP - EPT
        dummy = (jnp.arange(pad, dtype=jnp.int32) * 64) % N
        col = jnp.concatenate(
            [col.reshape(NW, EPT),
             jnp.broadcast_to(dummy, (NW, pad))], axis=1).reshape(-1)
        row = jnp.concatenate(
            [row.reshape(NW, EPT),
             jnp.broadcast_to(dummy, (NW, pad))], axis=1).reshape(-1)
        w = jnp.concatenate(
            [w.reshape(T, NW, EPT),
             jnp.zeros((T, NW, pad), jnp.float32)], axis=2).reshape(T, -1)
        return col, row, w

    return sc_layer, pad_edges


def kernel(x, edge_index, edge_weight, W_init, Wg, Ug, bg):
    T, N, D = x.shape
    E = edge_index.shape[1]
    L = W_init.shape[0]

    Qs = _evolve_weights(W_init, Wg, Ug, bg, T)
    row = edge_index[0].astype(jnp.int32)
    col = edge_index[1].astype(jnp.int32)
    w = edge_weight.astype(jnp.float32)

    sc_t, pad_edges = _make_sc_layer(T, N, E, D)
    colp, rowp, wp = pad_edges(col, row, w)

    hmm = _matmul_t(x, Qs[0])
    h = [hmm[t] for t in range(T)]
    for l in range(L):
        partials = [sc_t(h[t], colp, rowp, wp[t]) for t in range(T)]
        if l + 1 < L:
            h = [_combine_matmul(partials[t], Qs[l + 1, t]) for t in range(T)]
    return jnp.stack([_final_act(partials[t]) for t in range(T)])


# speedup vs baseline: 1.0062x; 1.0062x over previous
import functools

import jax
import jax.numpy as jnp
from jax import lax
from jax.experimental import pallas as pl
from jax.experimental.pallas import tpu as pltpu
from jax.experimental.pallas import tpu_sc as plsc

_SLOPE = (1.0 / 8.0 + 1.0 / 3.0) / 2.0

_NC = 2
_NS = 16
_LANES = 16


def _gru_body(L, T, wi_ref, wg_ref, ug_ref, bg_ref, q_ref):
    def sigmoid(v):
        return 1.0 / (1.0 + jnp.exp(-v))

    for l in range(L):
        Q = wi_ref[l]
        for t in range(T):
            z = Q
            upd = sigmoid(
                jnp.dot(wg_ref[l, 0], z, preferred_element_type=jnp.float32)
                + jnp.dot(ug_ref[l, 0], Q, preferred_element_type=jnp.float32)
                + bg_ref[l, 0])
            rst = sigmoid(
                jnp.dot(wg_ref[l, 1], z, preferred_element_type=jnp.float32)
                + jnp.dot(ug_ref[l, 1], Q, preferred_element_type=jnp.float32)
                + bg_ref[l, 1])
            hcap = jnp.tanh(
                jnp.dot(wg_ref[l, 2], z, preferred_element_type=jnp.float32)
                + jnp.dot(ug_ref[l, 2], rst * Q, preferred_element_type=jnp.float32)
                + bg_ref[l, 2])
            Q = (1.0 - upd) * Q + upd * hcap
            q_ref[l, t] = Q


def _evolve_weights(W_init, Wg, Ug, bg, T):
    L, D, _ = W_init.shape
    return pl.pallas_call(
        functools.partial(_gru_body, L, T),
        out_shape=jax.ShapeDtypeStruct((L, T, D, D), jnp.float32),
    )(W_init, Wg, Ug, bg)


def _mm_body(x_ref, q_ref, o_ref):
    o_ref[0] = jnp.dot(x_ref[0], q_ref[0], preferred_element_type=jnp.float32)


def _matmul_t(x, q):
    T, N, D = x.shape
    return pl.pallas_call(
        _mm_body,
        grid=(T,),
        in_specs=[
            pl.BlockSpec((1, N, D), lambda t: (t, 0, 0)),
            pl.BlockSpec((1, D, D), lambda t: (t, 0, 0)),
        ],
        out_specs=pl.BlockSpec((1, N, D), lambda t: (t, 0, 0)),
        out_shape=jax.ShapeDtypeStruct((T, N, D), jnp.float32),
    )(x, q)


def _combine_mm_body(p_ref, q_ref, o_ref):
    s = p_ref[0] + p_ref[1]
    a = jnp.where(s >= 0, s, s * _SLOPE)
    o_ref[...] = jnp.dot(a, q_ref[...], preferred_element_type=jnp.float32)


def _combine_matmul(partials, q):
    _, N, D = partials.shape
    NB = 5
    BN = N // NB
    return pl.pallas_call(
        _combine_mm_body,
        grid=(NB,),
        in_specs=[
            pl.BlockSpec((2, BN, D), lambda nb: (0, nb, 0)),
            pl.BlockSpec((D, D), lambda nb: (0, 0)),
        ],
        out_specs=pl.BlockSpec((BN, D), lambda nb: (nb, 0)),
        out_shape=jax.ShapeDtypeStruct((N, D), jnp.float32),
    )(partials, q)


def _final_body(p_ref, o_ref):
    s = p_ref[0] + p_ref[1]
    o_ref[...] = jnp.where(s >= 0, s, s * _SLOPE)


def _final_act(partials):
    _, N, D = partials.shape
    NB = 5
    BN = N // NB
    return pl.pallas_call(
        _final_body,
        grid=(NB,),
        in_specs=[pl.BlockSpec((2, BN, D), lambda nb: (0, nb, 0))],
        out_specs=pl.BlockSpec((BN, D), lambda nb: (nb, 0)),
        out_shape=jax.ShapeDtypeStruct((N, D), jnp.float32),
    )(partials)


def _make_sc_layer(T, N, E, D):
    NW = _NC * _NS
    EPT = E // NW
    K = 128
    NCHUNK = 79
    EPP = NCHUNK * K
    MAIN = NCHUNK - 1
    RPT = 624
    LAST = N - (_NS - 1) * RPT
    assert EPP >= EPT and MAIN % 3 == 0 and RPT % 8 == 0 and LAST % 8 == 0

    mesh = plsc.VectorSubcoreMesh(core_axis_name="c", subcore_axis_name="s")

    @functools.partial(
        pl.kernel,
        mesh=mesh,
        out_type=jax.ShapeDtypeStruct((_NC, N, D), jnp.float32),
        scratch_types=[
            pltpu.VMEM((K,), jnp.int32),
            pltpu.VMEM((K,), jnp.int32),
            pltpu.VMEM((K,), jnp.int32),
            pltpu.VMEM((K,), jnp.int32),
            pltpu.VMEM((K,), jnp.int32),
            pltpu.VMEM((K,), jnp.int32),
            pltpu.VMEM((K,), jnp.float32),
            pltpu.VMEM((K,), jnp.float32),
            pltpu.VMEM((K,), jnp.float32),
            pltpu.VMEM((K,), jnp.int32),
            pltpu.VMEM((K,), jnp.int32),
            pltpu.VMEM((K,), jnp.int32),
            pltpu.VMEM((K, D), jnp.float32),
            pltpu.VMEM((K, D), jnp.float32),
            pltpu.VMEM((K, D), jnp.float32),
            pltpu.VMEM_SHARED((N, D), jnp.float32),
            pltpu.SemaphoreType.DMA,
            pltpu.SemaphoreType.DMA,
            pltpu.SemaphoreType.DMA,
            pltpu.SemaphoreType.DMA,
            pltpu.SemaphoreType.DMA,
            pltpu.SemaphoreType.DMA,
            pltpu.SemaphoreType.DMA,
            pltpu.SemaphoreType.DMA,
            pltpu.SemaphoreType.DMA,
        ],
    )
    def sc_layer(h, col_hbm, row_hbm, w_hbm, out_hbm,
                 cb0, cb1, cb2, rb0, rb1, rb2, wb0, wb1, wb2,
                 srb0, srb1, srb2, rows0, rows1, rows2, acc,
                 g0, g1, g2, i0, i1, i2, s0, s1, s2):
        c = lax.axis_index("c")
        s = lax.axis_index("s")
        wid = c * _NS + s
        base_e = wid * EPP
        CB = (cb0, cb1, cb2)
        RB = (rb0, rb1, rb2)
        WB = (wb0, wb1, wb2)
        SRB = (srb0, srb1, srb2)
        ROWS = (rows0, rows1, rows2)
        GS = (g0, g1, g2)
        IS = (i0, i1, i2)
        SS = (s0, s1, s2)
        zero = jnp.zeros((_LANES,), jnp.float32)

        def issue_idx(i, b, sync):
            e0 = pl.multiple_of(base_e + i * K, 8)
            if sync:
                pltpu.sync_copy(col_hbm.at[pl.ds(e0, K)], CB[b])
                pltpu.sync_copy(row_hbm.at[pl.ds(e0, K)], RB[b])
                pltpu.sync_copy(w_hbm.at[pl.ds(e0, K)], WB[b])
            else:
                pltpu.async_copy(col_hbm.at[pl.ds(e0, K)], CB[b], IS[b])
                pltpu.async_copy(row_hbm.at[pl.ds(e0, K)], RB[b], IS[b])
                pltpu.async_copy(w_hbm.at[pl.ds(e0, K)], WB[b], IS[b])

        def wait_idx(i, b):
            e0 = pl.multiple_of(base_e + i * K, 8)
            pltpu.make_async_copy(col_hbm.at[pl.ds(e0, K)], CB[b], IS[b]).wait()
            pltpu.make_async_copy(row_hbm.at[pl.ds(e0, K)], RB[b], IS[b]).wait()
            pltpu.make_async_copy(w_hbm.at[pl.ds(e0, K)], WB[b], IS[b]).wait()

        def issue_gather(b):
            pltpu.async_copy(h.at[CB[b]], ROWS[b], GS[b])

        def drain_scatter(b):
            pltpu.make_async_copy(ROWS[b], acc.at[SRB[b]], SS[b]).wait()

        def finish_chunk(b):
            pltpu.make_async_copy(h.at[CB[b]], ROWS[b], GS[b]).wait()
            rbuf = ROWS[b]

            for g in range(K // _LANES):
                sl = pl.ds(g * _LANES, _LANES)
                SRB[b][sl] = RB[b][sl]

            def scale(g, _):
                wg = WB[b][pl.ds(g * _LANES, _LANES)]
                for jj in range(_LANES):
                    wj = wg[jj]
                    r = g * _LANES + jj
                    for d in range(D // _LANES):
                        sl = pl.ds(d * _LANES, _LANES)
                        rbuf[r, sl] = rbuf[r, sl] * wj
                return 0

            lax.fori_loop(0, K // _LANES, scale, 0)
            pltpu.async_copy(rbuf, acc.at[SRB[b]], SS[b], add=True)

        if True:
            def zfill(r, _):
                for d in range(D // _LANES):
                    rows0[r, pl.ds(d * _LANES, _LANES)] = zero
                return 0

            lax.fori_loop(0, K, zfill, 0)
            for off in range(0, RPT - RPT % K, K):
                pltpu.sync_copy(rows0, acc.at[pl.ds(s * RPT + off, K)])
            if RPT % K:
                pltpu.sync_copy(
                    rows0.at[pl.ds(0, RPT % K)],
                    acc.at[pl.ds(s * RPT + RPT - RPT % K, RPT % K)])

            @pl.when(s == _NS - 1)
            def _():
                pltpu.sync_copy(rows0.at[pl.ds(0, LAST - RPT)],
                                acc.at[pl.ds(N - (LAST - RPT), LAST - RPT)])

            plsc.subcore_barrier()

            issue_idx(0, 0, True)
            issue_gather(0)
            issue_idx(1, 1, True)
            issue_gather(1)
            issue_idx(2, 2, False)

            def triple(p, _):
                for u in range(3):
                    i = p * 3 + u
                    nb = (u + 2) % 3
                    @pl.when(i + 2 < NCHUNK)
                    def _():
                        wait_idx(i + 2, nb)

                    @pl.when(i >= 1)
                    def _():
                        drain_scatter(nb)

                    @pl.when(i + 2 < NCHUNK)
                    def _():
                        issue_gather(nb)

                    finish_chunk(u)
                    nxt = i + 3

                    @pl.when(nxt < NCHUNK)
                    def _():
                        issue_idx(nxt, u, False)
                return 0

            lax.fori_loop(0, MAIN // 3, triple, 0)
            finish_chunk(MAIN % 3)
            drain_scatter((MAIN - 1) % 3)
            drain_scatter(MAIN % 3)
            plsc.subcore_barrier()

            pltpu.sync_copy(acc.at[pl.ds(s * RPT, RPT)],
                            out_hbm.at[c, pl.ds(s * RPT, RPT)])

            @pl.when(s == _NS - 1)
            def _():
                pltpu.sync_copy(acc.at[pl.ds(N - (LAST - RPT), LAST - RPT)],
                                out_hbm.at[c, pl.ds(N - (LAST - RPT),
                                                    LAST - RPT)])

            plsc.subcore_barrier()

    def pad_edges(col, row, w):
        pad = EPP - EPT
        dummy = (jnp.arange(pad, dtype=jnp.int32) * 64) % N
        col = jnp.concatenate(
            [col.reshape(NW, EPT),
             jnp.broadcast_to(dummy, (NW, pad))], axis=1).reshape(-1)
        row = jnp.concatenate(
            [row.reshape(NW, EPT),
             jnp.broadcast_to(dummy, (NW, pad))], axis=1).reshape(-1)
        w = jnp.concatenate(
            [w.reshape(T, NW, EPT),
             jnp.zeros((T, NW, pad), jnp.float32)], axis=2).reshape(T, -1)
        return col, row, w

    return sc_layer, pad_edges


def kernel(x, edge_index, edge_weight, W_init, Wg, Ug, bg):
    T, N, D = x.shape
    E = edge_index.shape[1]
    L = W_init.shape[0]

    Qs = _evolve_weights(W_init, Wg, Ug, bg, T)
    row = edge_index[0].astype(jnp.int32)
    col = edge_index[1].astype(jnp.int32)
    w = edge_weight.astype(jnp.float32)

    sc_t, pad_edges = _make_sc_layer(T, N, E, D)
    colp, rowp, wp = pad_edges(col, row, w)

    hmm = _matmul_t(x, Qs[0])
    h = [hmm[t] for t in range(T)]
    for l in range(L):
        partials = [sc_t(h[t], colp, rowp, wp[t]) for t in range(T)]
        if l + 1 < L:
            h = [_combine_matmul(partials[t], Qs[l + 1, t]) for t in range(T)]
    return jnp.stack([_final_act(partials[t]) for t in range(T)])
